# baseline (speedup 1.0000x reference)
import functools

import jax
import jax.numpy as jnp
from jax import lax
from jax.experimental import pallas as pl
from jax.experimental.pallas import tpu as pltpu
from jax.experimental.pallas import tpu_sc as plsc

NUM_WORKERS = 32
CHUNK = 2048
GATHER_B = 2048
LANES = 16


def _make_sc_call(n_pad, nx, ny, nz):
    ppw = n_pad // NUM_WORKERS
    n_chunks = ppw // CHUNK
    sx = ny * nz
    sy = nz

    mesh = plsc.VectorSubcoreMesh(core_axis_name="c", subcore_axis_name="s")

    scratch = (
        [pltpu.VMEM((CHUNK,), jnp.float32) for _ in range(3)]
        + [pltpu.VMEM((CHUNK,), jnp.float32) for _ in range(3)]
        + [pltpu.VMEM((CHUNK,), jnp.int32) for _ in range(16)]
        + [pltpu.VMEM((CHUNK,), jnp.float32) for _ in range(16)]
        + [pltpu.VMEM((CHUNK,), jnp.float32) for _ in range(2)]
        + [pltpu.VMEM((LANES,), jnp.float32) for _ in range(6)]
        + [pltpu.SemaphoreType.DMA]
    )

    @functools.partial(
        pl.kernel,
        mesh=mesh,
        out_type=(
            jax.ShapeDtypeStruct((n_pad,), jnp.float32),
            jax.ShapeDtypeStruct((n_pad,), jnp.float32),
        ),
        scratch_types=scratch,
    )
    def sc_call(posx_h, posy_h, posz_h, par_h, tab_h,
                outm_h, outs_h, *refs):
        pos_v = refs[0:3]
        frac_v = refs[3:6]
        idx_v = refs[6:22]
        res_v = refs[22:38]
        out_v = refs[38:40]
        par_v = refs[40:46]
        sem = refs[46]

        wid = lax.axis_index("s") * 2 + lax.axis_index("c")
        base_w = wid * ppw

        for d in range(6):
            pltpu.sync_copy(par_h.at[pl.ds(d * LANES, LANES)], par_v[d])
        minx = par_v[0][:]
        miny = par_v[1][:]
        minz = par_v[2][:]
        sclx = par_v[3][:]
        scly = par_v[4][:]
        sclz = par_v[5][:]

        def chunk_body(t, carry):
            base = base_w + t * CHUNK
            pltpu.sync_copy(posx_h.at[pl.ds(base, CHUNK)], pos_v[0])
            pltpu.sync_copy(posy_h.at[pl.ds(base, CHUNK)], pos_v[1])
            pltpu.sync_copy(posz_h.at[pl.ds(base, CHUNK)], pos_v[2])

            def index_body(g, c):
                sl = pl.ds(g * LANES, LANES)
                fx = jnp.maximum((pos_v[0][sl] - minx) * sclx, 0.0)
                fy = jnp.maximum((pos_v[1][sl] - miny) * scly, 0.0)
                fz = jnp.maximum((pos_v[2][sl] - minz) * sclz, 0.0)
                x0 = jnp.minimum(fx.astype(jnp.int32), nx - 2)
                y0 = jnp.minimum(fy.astype(jnp.int32), ny - 2)
                z0 = jnp.minimum(fz.astype(jnp.int32), nz - 2)
                frac_v[0][sl] = fx - x0.astype(jnp.float32)
                frac_v[1][sl] = fy - y0.astype(jnp.float32)
                frac_v[2][sl] = fz - z0.astype(jnp.float32)
                b = (x0 * sx + y0 * sy + z0) * 2
                for k, off in enumerate(
                        (0, 1, sy, sy + 1, sx, sx + 1, sx + sy, sx + sy + 1)):
                    idx_v[k][sl] = b + (2 * off)
                    idx_v[8 + k][sl] = b + (2 * off + 1)
                return c

            lax.fori_loop(0, CHUNK // LANES, index_body, 0)

            cps = [pltpu.async_copy(tab_h.at[idx_v[k]], res_v[k], sem)
                   for k in range(16)]
            for cp in cps:
                cp.wait()

            def combine_body(g, c):
                sl = pl.ds(g * LANES, LANES)
                tx = frac_v[0][sl]
                ty = frac_v[1][sl]
                tz = frac_v[2][sl]
                ux = 1.0 - tx
                uy = 1.0 - ty
                uz = 1.0 - tz
                c00 = uy * uz
                c01 = uy * tz
                c10 = ty * uz
                c11 = ty * tz
                w0 = ux * c00
                w1 = ux * c01
                w2 = ux * c10
                w3 = ux * c11
                w4 = tx * c00
                w5 = tx * c01
                w6 = tx * c10
                w7 = tx * c11
                m = (w0 * res_v[0][sl] + w1 * res_v[1][sl]
                     + w2 * res_v[2][sl] + w3 * res_v[3][sl]
                     + w4 * res_v[4][sl] + w5 * res_v[5][sl]
                     + w6 * res_v[6][sl] + w7 * res_v[7][sl])
                s = (w0 * res_v[8][sl] + w1 * res_v[9][sl]
                     + w2 * res_v[10][sl] + w3 * res_v[11][sl]
                     + w4 * res_v[12][sl] + w5 * res_v[13][sl]
                     + w6 * res_v[14][sl] + w7 * res_v[15][sl])
                out_v[0][sl] = m
                out_v[1][sl] = s
                return c

            lax.fori_loop(0, CHUNK // LANES, combine_body, 0)

            pltpu.sync_copy(out_v[0], outm_h.at[pl.ds(base, CHUNK)])
            pltpu.sync_copy(out_v[1], outs_h.at[pl.ds(base, CHUNK)])
            return carry

        lax.fori_loop(0, n_chunks, chunk_body, 0)

    return sc_call


def kernel(pos, grid, min_bound, max_bound):
    n = pos.shape[0]
    _, nx, ny, nz = grid.shape

    tile = NUM_WORKERS * CHUNK
    n_pad = -(-n // tile) * tile
    pad = n_pad - n

    posx = pos[:, 0]
    posy = pos[:, 1]
    posz = pos[:, 2]
    if pad:
        posx = jnp.concatenate([posx, posx[:pad]])
        posy = jnp.concatenate([posy, posy[:pad]])
        posz = jnp.concatenate([posz, posz[:pad]])

    grid_range = jnp.clip(max_bound - min_bound, 1e-6, None)
    dims = jnp.array([nx - 1, ny - 1, nz - 1], dtype=jnp.float32)
    scales = dims / grid_range
    params = jnp.concatenate(
        [
            jnp.repeat(min_bound.astype(jnp.float32), LANES),
            jnp.repeat(scales.astype(jnp.float32), LANES),
        ]
    )

    table = jnp.transpose(grid, (1, 2, 3, 0)).reshape(-1)

    sc_call = _make_sc_call(n_pad, nx, ny, nz)
    outm, outs = sc_call(posx, posy, posz, params, table)
    return (outm[:n], outs[:n])


# speedup vs baseline: 26.3044x; 26.3044x over previous
import functools

import jax
import jax.numpy as jnp
from jax import lax
from jax.experimental import pallas as pl
from jax.experimental.pallas import tpu as pltpu
from jax.experimental.pallas import tpu_sc as plsc

NUM_WORKERS = 32
CHUNK = 2048
LANES = 16
NBUF = 2


def _make_sc_call(n_pad, nx, ny, nz):
    ppw = n_pad // NUM_WORKERS
    n_chunks = ppw // CHUNK
    sx = ny * nz
    sy = nz

    mesh = plsc.VectorSubcoreMesh(core_axis_name="c", subcore_axis_name="s")

    bank_scratch = (
        [pltpu.VMEM((CHUNK,), jnp.float32) for _ in range(3)]
        + [pltpu.VMEM((CHUNK,), jnp.float32) for _ in range(3)]
        + [pltpu.VMEM((CHUNK,), jnp.int32) for _ in range(8)]
        + [pltpu.VMEM((CHUNK,), jnp.float32) for _ in range(16)]
        + [pltpu.SemaphoreType.DMA]
    )
    scratch = (
        bank_scratch * NBUF
        + [pltpu.VMEM((CHUNK,), jnp.float32) for _ in range(2)]
        + [pltpu.VMEM((LANES,), jnp.float32) for _ in range(6)]
    )

    @functools.partial(
        pl.kernel,
        mesh=mesh,
        out_type=(
            jax.ShapeDtypeStruct((n_pad,), jnp.float32),
            jax.ShapeDtypeStruct((n_pad,), jnp.float32),
        ),
        scratch_types=scratch,
    )
    def sc_call(posx_h, posy_h, posz_h, par_h, g0_h, g1_h,
                outm_h, outs_h, *refs):
        banks = []
        for b in range(NBUF):
            r = refs[b * 31:(b + 1) * 31]
            banks.append(dict(pos=r[0:3], frac=r[3:6], idx=r[6:14],
                              res=r[14:30], sem=r[30]))
        out_v = refs[62:64]
        par_v = refs[64:70]

        wid = lax.axis_index("s") * 2 + lax.axis_index("c")
        base_w = wid * ppw

        for d in range(6):
            pltpu.sync_copy(par_h.at[pl.ds(d * LANES, LANES)], par_v[d])
        minx = par_v[0][:]
        miny = par_v[1][:]
        minz = par_v[2][:]
        sclx = par_v[3][:]
        scly = par_v[4][:]
        sclz = par_v[5][:]

        def load_and_index(t, bk):
            base = base_w + t * CHUNK
            pltpu.sync_copy(posx_h.at[pl.ds(base, CHUNK)], bk["pos"][0])
            pltpu.sync_copy(posy_h.at[pl.ds(base, CHUNK)], bk["pos"][1])
            pltpu.sync_copy(posz_h.at[pl.ds(base, CHUNK)], bk["pos"][2])

            def index_body(g, c):
                sl = pl.ds(g * LANES, LANES)
                fx = jnp.maximum((bk["pos"][0][sl] - minx) * sclx, 0.0)
                fy = jnp.maximum((bk["pos"][1][sl] - miny) * scly, 0.0)
                fz = jnp.maximum((bk["pos"][2][sl] - minz) * sclz, 0.0)
                x0 = jnp.minimum(fx.astype(jnp.int32), nx - 2)
                y0 = jnp.minimum(fy.astype(jnp.int32), ny - 2)
                z0 = jnp.minimum(fz.astype(jnp.int32), nz - 2)
                bk["frac"][0][sl] = fx - x0.astype(jnp.float32)
                bk["frac"][1][sl] = fy - y0.astype(jnp.float32)
                bk["frac"][2][sl] = fz - z0.astype(jnp.float32)
                b = x0 * sx + y0 * sy + z0
                bk["idx"][0][sl] = b
                bk["idx"][1][sl] = b + 1
                bk["idx"][2][sl] = b + sy
                bk["idx"][3][sl] = b + (sy + 1)
                bk["idx"][4][sl] = b + sx
                bk["idx"][5][sl] = b + (sx + 1)
                bk["idx"][6][sl] = b + (sx + sy)
                bk["idx"][7][sl] = b + (sx + sy + 1)
                return c

            lax.fori_loop(0, CHUNK // LANES, index_body, 0)

        def gathers(bk):
            return (
                [pltpu.make_async_copy(g0_h.at[bk["idx"][k]], bk["res"][k],
                                       bk["sem"]) for k in range(8)]
                + [pltpu.make_async_copy(g1_h.at[bk["idx"][k]],
                                         bk["res"][8 + k], bk["sem"])
                   for k in range(8)]
            )

        def fire(bk):
            for cp in gathers(bk):
                cp.start()

        def drain(bk):
            for cp in gathers(bk):
                cp.wait()

        def combine_store(t, bk):
            base = base_w + t * CHUNK
            res_v = bk["res"]

            def combine_body(g, c):
                sl = pl.ds(g * LANES, LANES)
                tx = bk["frac"][0][sl]
                ty = bk["frac"][1][sl]
                tz = bk["frac"][2][sl]
                ux = 1.0 - tx
                uy = 1.0 - ty
                uz = 1.0 - tz
                c00 = uy * uz
                c01 = uy * tz
                c10 = ty * uz
                c11 = ty * tz
                w0 = ux * c00
                w1 = ux * c01
                w2 = ux * c10
                w3 = ux * c11
                w4 = tx * c00
                w5 = tx * c01
                w6 = tx * c10
                w7 = tx * c11
                m = (w0 * res_v[0][sl] + w1 * res_v[1][sl]
                     + w2 * res_v[2][sl] + w3 * res_v[3][sl]
                     + w4 * res_v[4][sl] + w5 * res_v[5][sl]
                     + w6 * res_v[6][sl] + w7 * res_v[7][sl])
                s = (w0 * res_v[8][sl] + w1 * res_v[9][sl]
                     + w2 * res_v[10][sl] + w3 * res_v[11][sl]
                     + w4 * res_v[12][sl] + w5 * res_v[13][sl]
                     + w6 * res_v[14][sl] + w7 * res_v[15][sl])
                out_v[0][sl] = m
                out_v[1][sl] = s
                return c

            lax.fori_loop(0, CHUNK // LANES, combine_body, 0)
            pltpu.sync_copy(out_v[0], outm_h.at[pl.ds(base, CHUNK)])
            pltpu.sync_copy(out_v[1], outs_h.at[pl.ds(base, CHUNK)])

        load_and_index(0, banks[0])
        fire(banks[0])

        def pair_body(p, carry):
            t0 = 2 * p
            t1 = t0 + 1
            t2 = t0 + 2
            load_and_index(t1, banks[1])
            fire(banks[1])
            drain(banks[0])
            combine_store(t0, banks[0])

            @pl.when(t2 < n_chunks)
            def _():
                load_and_index(t2, banks[0])
                fire(banks[0])

            drain(banks[1])
            combine_store(t1, banks[1])
            return carry

        lax.fori_loop(0, n_chunks // 2, pair_body, 0)

    return sc_call


def kernel(pos, grid, min_bound, max_bound):
    n = pos.shape[0]
    _, nx, ny, nz = grid.shape

    tile = NUM_WORKERS * CHUNK * 2
    n_pad = -(-n // tile) * tile
    pad = n_pad - n

    posx = pos[:, 0]
    posy = pos[:, 1]
    posz = pos[:, 2]
    if pad:
        reps = -(-pad // n)
        posx = jnp.concatenate([posx] + [posx[:pad]] * reps)[:n_pad]
        posy = jnp.concatenate([posy] + [posy[:pad]] * reps)[:n_pad]
        posz = jnp.concatenate([posz] + [posz[:pad]] * reps)[:n_pad]

    grid_range = jnp.clip(max_bound - min_bound, 1e-6, None)
    dims = jnp.array([nx - 1, ny - 1, nz - 1], dtype=jnp.float32)
    scales = dims / grid_range
    params = jnp.concatenate(
        [
            jnp.repeat(min_bound.astype(jnp.float32), LANES),
            jnp.repeat(scales.astype(jnp.float32), LANES),
        ]
    )

    g0 = grid[0].reshape(-1)
    g1 = grid[1].reshape(-1)

    sc_call = _make_sc_call(n_pad, nx, ny, nz)
    outm, outs = sc_call(posx, posy, posz, params, g0, g1)
    return (outm[:n], outs[:n])


# speedup vs baseline: 26.6548x; 1.0133x over previous
import functools

import jax
import jax.numpy as jnp
from jax import lax
from jax.experimental import pallas as pl
from jax.experimental.pallas import tpu as pltpu
from jax.experimental.pallas import tpu_sc as plsc

NUM_WORKERS = 32
CHUNK = 2048
LANES = 16
NBUF = 2


def _make_sc_call(n, nx, ny, nz):
    n_chunks = -(-n // CHUNK)
    total_chunks = -(-n_chunks // (2 * NUM_WORKERS)) * (2 * NUM_WORKERS)
    cpw = total_chunks // NUM_WORKERS
    last_base = n - CHUNK
    sx = ny * nz
    sy = nz

    mesh = plsc.VectorSubcoreMesh(core_axis_name="c", subcore_axis_name="s")

    bank_scratch = (
        [pltpu.VMEM((CHUNK,), jnp.float32) for _ in range(3)]
        + [pltpu.VMEM((CHUNK,), jnp.float32) for _ in range(3)]
        + [pltpu.VMEM((CHUNK,), jnp.int32) for _ in range(8)]
        + [pltpu.VMEM((CHUNK,), jnp.float32) for _ in range(16)]
        + [pltpu.SemaphoreType.DMA]
    )
    scratch = (
        bank_scratch * NBUF
        + [pltpu.VMEM((CHUNK,), jnp.float32) for _ in range(2)]
        + [pltpu.VMEM((LANES,), jnp.float32) for _ in range(6)]
    )

    @functools.partial(
        pl.kernel,
        mesh=mesh,
        out_type=(
            jax.ShapeDtypeStruct((n,), jnp.float32),
            jax.ShapeDtypeStruct((n,), jnp.float32),
        ),
        scratch_types=scratch,
    )
    def sc_call(posx_h, posy_h, posz_h, par_h, g0_h, g1_h,
                outm_h, outs_h, *refs):
        nb = 31
        banks = []
        for b in range(NBUF):
            r = refs[b * nb:(b + 1) * nb]
            banks.append(dict(pos=r[0:3], frac=r[3:6], idx=r[6:14],
                              res=r[14:30], sem=r[30]))
        out_v = refs[2 * nb:2 * nb + 2]
        par_v = refs[2 * nb + 2:2 * nb + 8]

        wid = lax.axis_index("s") * 2 + lax.axis_index("c")

        for d in range(6):
            pltpu.sync_copy(par_h.at[pl.ds(d * LANES, LANES)], par_v[d])
        minx = par_v[0][:]
        miny = par_v[1][:]
        minz = par_v[2][:]
        sclx = par_v[3][:]
        scly = par_v[4][:]
        sclz = par_v[5][:]
        def chunk_base(j):
            t = j * NUM_WORKERS + wid
            return jnp.minimum(t * CHUNK, last_base)

        def load_and_index(j, bk):
            base = chunk_base(j)
            pltpu.sync_copy(posx_h.at[pl.ds(base, CHUNK)], bk["pos"][0])
            pltpu.sync_copy(posy_h.at[pl.ds(base, CHUNK)], bk["pos"][1])
            pltpu.sync_copy(posz_h.at[pl.ds(base, CHUNK)], bk["pos"][2])

            def index_body(g, c):
                sl = pl.ds(g * LANES, LANES)
                fx = jnp.maximum((bk["pos"][0][sl] - minx) * sclx, 0.0)
                fy = jnp.maximum((bk["pos"][1][sl] - miny) * scly, 0.0)
                fz = jnp.maximum((bk["pos"][2][sl] - minz) * sclz, 0.0)
                x0 = jnp.minimum(fx.astype(jnp.int32), nx - 2)
                y0 = jnp.minimum(fy.astype(jnp.int32), ny - 2)
                z0 = jnp.minimum(fz.astype(jnp.int32), nz - 2)
                bk["frac"][0][sl] = fx - x0.astype(jnp.float32)
                bk["frac"][1][sl] = fy - y0.astype(jnp.float32)
                bk["frac"][2][sl] = fz - z0.astype(jnp.float32)
                b = x0 * sx + y0 * sy + z0
                bk["idx"][0][sl] = b
                bk["idx"][1][sl] = b + 1
                bk["idx"][2][sl] = b + sy
                bk["idx"][3][sl] = b + (sy + 1)
                bk["idx"][4][sl] = b + sx
                bk["idx"][5][sl] = b + (sx + 1)
                bk["idx"][6][sl] = b + (sx + sy)
                bk["idx"][7][sl] = b + (sx + sy + 1)
                return c

            lax.fori_loop(0, CHUNK // LANES, index_body, 0)

        def gathers(bk):
            return (
                [pltpu.make_async_copy(g0_h.at[bk["idx"][k]], bk["res"][k],
                                       bk["sem"]) for k in range(8)]
                + [pltpu.make_async_copy(g1_h.at[bk["idx"][k]],
                                         bk["res"][8 + k], bk["sem"])
                   for k in range(8)]
            )

        def fire(bk):
            for cp in gathers(bk):
                cp.start()

        def drain(bk):
            for cp in gathers(bk):
                cp.wait()

        def combine_store(j, bk):
            base = chunk_base(j)
            res_v = bk["res"]

            def combine_body(g, c):
                sl = pl.ds(g * LANES, LANES)
                tx = bk["frac"][0][sl]
                ty = bk["frac"][1][sl]
                tz = bk["frac"][2][sl]
                ux = 1.0 - tx
                uy = 1.0 - ty
                uz = 1.0 - tz
                c00 = uy * uz
                c01 = uy * tz
                c10 = ty * uz
                c11 = ty * tz
                w0 = ux * c00
                w1 = ux * c01
                w2 = ux * c10
                w3 = ux * c11
                w4 = tx * c00
                w5 = tx * c01
                w6 = tx * c10
                w7 = tx * c11
                m = (w0 * res_v[0][sl] + w1 * res_v[1][sl]
                     + w2 * res_v[2][sl] + w3 * res_v[3][sl]
                     + w4 * res_v[4][sl] + w5 * res_v[5][sl]
                     + w6 * res_v[6][sl] + w7 * res_v[7][sl])
                s = (w0 * res_v[8][sl] + w1 * res_v[9][sl]
                     + w2 * res_v[10][sl] + w3 * res_v[11][sl]
                     + w4 * res_v[12][sl] + w5 * res_v[13][sl]
                     + w6 * res_v[14][sl] + w7 * res_v[15][sl])
                out_v[0][sl] = m
                out_v[1][sl] = s
                return c

            lax.fori_loop(0, CHUNK // LANES, combine_body, 0)
            pltpu.sync_copy(out_v[0], outm_h.at[pl.ds(base, CHUNK)])
            pltpu.sync_copy(out_v[1], outs_h.at[pl.ds(base, CHUNK)])

        load_and_index(0, banks[0])
        fire(banks[0])

        def pair_body(p, carry):
            j0 = 2 * p
            j1 = j0 + 1
            j2 = j0 + 2
            load_and_index(j1, banks[1])
            fire(banks[1])
            drain(banks[0])
            combine_store(j0, banks[0])

            @pl.when(j2 < cpw)
            def _():
                load_and_index(j2, banks[0])
                fire(banks[0])

            drain(banks[1])
            combine_store(j1, banks[1])
            return carry

        lax.fori_loop(0, cpw // 2, pair_body, 0)

    return sc_call


def kernel(pos, grid, min_bound, max_bound):
    n = pos.shape[0]
    _, nx, ny, nz = grid.shape

    n_al = -(-n // 8) * 8
    if n_al != n:
        pos = jnp.concatenate([pos, pos[: n_al - n]])

    posx = pos[:, 0]
    posy = pos[:, 1]
    posz = pos[:, 2]

    grid_range = jnp.clip(max_bound - min_bound, 1e-6, None)
    dims = jnp.array([nx - 1, ny - 1, nz - 1], dtype=jnp.float32)
    scales = dims / grid_range
    params = jnp.concatenate(
        [
            jnp.repeat(min_bound.astype(jnp.float32), LANES),
            jnp.repeat(scales.astype(jnp.float32), LANES),
        ]
    )

    g0 = grid[0].reshape(-1)
    g1 = grid[1].reshape(-1)

    sc_call = _make_sc_call(n_al, nx, ny, nz)
    outm, outs = sc_call(posx, posy, posz, params, g0, g1)
    if n_al != n:
        return (outm[:n], outs[:n])
    return (outm, outs)


# speedup vs baseline: 29.7901x; 1.1176x over previous
import functools

import jax
import jax.numpy as jnp
from jax import lax
from jax.experimental import pallas as pl
from jax.experimental.pallas import tpu as pltpu
from jax.experimental.pallas import tpu_sc as plsc

NUM_WORKERS = 32
CHUNK = 2048
LANES = 16
NBUF = 2


def _make_sc_call(n, nx, ny, nz):
    n_chunks = -(-n // CHUNK)
    total_chunks = -(-n_chunks // (2 * NUM_WORKERS)) * (2 * NUM_WORKERS)
    cpw = total_chunks // NUM_WORKERS
    last_base = n - CHUNK
    sx = ny * nz
    sy = nz

    mesh = plsc.VectorSubcoreMesh(core_axis_name="c", subcore_axis_name="s")

    bank_scratch = (
        [pltpu.VMEM((CHUNK,), jnp.float32) for _ in range(3)]
        + [pltpu.VMEM((CHUNK,), jnp.float32) for _ in range(3)]
        + [pltpu.VMEM((CHUNK,), jnp.int32) for _ in range(8)]
        + [pltpu.VMEM((CHUNK,), jnp.float32) for _ in range(16)]
        + [pltpu.SemaphoreType.DMA]
    )
    scratch = (
        bank_scratch * NBUF
        + [pltpu.VMEM((CHUNK,), jnp.float32) for _ in range(2)]
        + [pltpu.VMEM((LANES,), jnp.float32) for _ in range(6)]
    )

    @functools.partial(
        pl.kernel,
        mesh=mesh,
        out_type=(
            jax.ShapeDtypeStruct((n,), jnp.float32),
            jax.ShapeDtypeStruct((n,), jnp.float32),
        ),
        scratch_types=scratch,
    )
    def sc_call(posx_h, posy_h, posz_h, par_h, tab_h,
                outm_h, outs_h, *refs):
        g0_h = tab_h.at[pl.ds(0, sx * nx)]
        g1_h = tab_h.at[pl.ds(sx * nx, sx * nx)]
        nb = 31
        banks = []
        for b in range(NBUF):
            r = refs[b * nb:(b + 1) * nb]
            banks.append(dict(pos=r[0:3], frac=r[3:6], idx=r[6:14],
                              res=r[14:30], sem=r[30]))
        out_v = refs[2 * nb:2 * nb + 2]
        par_v = refs[2 * nb + 2:2 * nb + 8]

        wid = lax.axis_index("s") * 2 + lax.axis_index("c")

        for d in range(6):
            pltpu.sync_copy(par_h.at[pl.ds(d * LANES, LANES)], par_v[d])
        minx = par_v[0][:]
        miny = par_v[1][:]
        minz = par_v[2][:]
        sclx = par_v[3][:]
        scly = par_v[4][:]
        sclz = par_v[5][:]
        def chunk_base(j):
            t = j * NUM_WORKERS + wid
            return jnp.minimum(t * CHUNK, last_base)

        def load_and_index(j, bk):
            base = chunk_base(j)
            pltpu.sync_copy(posx_h.at[pl.ds(base, CHUNK)], bk["pos"][0])
            pltpu.sync_copy(posy_h.at[pl.ds(base, CHUNK)], bk["pos"][1])
            pltpu.sync_copy(posz_h.at[pl.ds(base, CHUNK)], bk["pos"][2])

            def index_body(g, c):
                sl = pl.ds(g * LANES, LANES)
                fx = jnp.maximum((bk["pos"][0][sl] - minx) * sclx, 0.0)
                fy = jnp.maximum((bk["pos"][1][sl] - miny) * scly, 0.0)
                fz = jnp.maximum((bk["pos"][2][sl] - minz) * sclz, 0.0)
                x0 = jnp.minimum(fx.astype(jnp.int32), nx - 2)
                y0 = jnp.minimum(fy.astype(jnp.int32), ny - 2)
                z0 = jnp.minimum(fz.astype(jnp.int32), nz - 2)
                bk["frac"][0][sl] = fx - x0.astype(jnp.float32)
                bk["frac"][1][sl] = fy - y0.astype(jnp.float32)
                bk["frac"][2][sl] = fz - z0.astype(jnp.float32)
                b = x0 * sx + y0 * sy + z0
                bk["idx"][0][sl] = b
                bk["idx"][1][sl] = b + 1
                bk["idx"][2][sl] = b + sy
                bk["idx"][3][sl] = b + (sy + 1)
                bk["idx"][4][sl] = b + sx
                bk["idx"][5][sl] = b + (sx + 1)
                bk["idx"][6][sl] = b + (sx + sy)
                bk["idx"][7][sl] = b + (sx + sy + 1)
                return c

            lax.fori_loop(0, CHUNK // LANES, index_body, 0)

        def gathers(bk):
            return (
                [pltpu.make_async_copy(g0_h.at[bk["idx"][k]], bk["res"][k],
                                       bk["sem"]) for k in range(8)]
                + [pltpu.make_async_copy(g1_h.at[bk["idx"][k]],
                                         bk["res"][8 + k], bk["sem"])
                   for k in range(8)]
            )

        def fire(bk):
            for cp in gathers(bk):
                cp.start()

        def drain(bk):
            for cp in gathers(bk):
                cp.wait()

        def combine_store(j, bk):
            base = chunk_base(j)
            res_v = bk["res"]

            def combine_body(g, c):
                sl = pl.ds(g * LANES, LANES)
                tx = bk["frac"][0][sl]
                ty = bk["frac"][1][sl]
                tz = bk["frac"][2][sl]
                ux = 1.0 - tx
                uy = 1.0 - ty
                uz = 1.0 - tz
                c00 = uy * uz
                c01 = uy * tz
                c10 = ty * uz
                c11 = ty * tz
                w0 = ux * c00
                w1 = ux * c01
                w2 = ux * c10
                w3 = ux * c11
                w4 = tx * c00
                w5 = tx * c01
                w6 = tx * c10
                w7 = tx * c11
                m = (w0 * res_v[0][sl] + w1 * res_v[1][sl]
                     + w2 * res_v[2][sl] + w3 * res_v[3][sl]
                     + w4 * res_v[4][sl] + w5 * res_v[5][sl]
                     + w6 * res_v[6][sl] + w7 * res_v[7][sl])
                s = (w0 * res_v[8][sl] + w1 * res_v[9][sl]
                     + w2 * res_v[10][sl] + w3 * res_v[11][sl]
                     + w4 * res_v[12][sl] + w5 * res_v[13][sl]
                     + w6 * res_v[14][sl] + w7 * res_v[15][sl])
                out_v[0][sl] = m
                out_v[1][sl] = s
                return c

            lax.fori_loop(0, CHUNK // LANES, combine_body, 0)
            pltpu.sync_copy(out_v[0], outm_h.at[pl.ds(base, CHUNK)])
            pltpu.sync_copy(out_v[1], outs_h.at[pl.ds(base, CHUNK)])

        load_and_index(0, banks[0])
        fire(banks[0])

        def pair_body(p, carry):
            j0 = 2 * p
            j1 = j0 + 1
            j2 = j0 + 2
            load_and_index(j1, banks[1])
            fire(banks[1])
            drain(banks[0])
            combine_store(j0, banks[0])

            @pl.when(j2 < cpw)
            def _():
                load_and_index(j2, banks[0])
                fire(banks[0])

            drain(banks[1])
            combine_store(j1, banks[1])
            return carry

        lax.fori_loop(0, cpw // 2, pair_body, 0)

    return sc_call


def kernel(pos, grid, min_bound, max_bound):
    n = pos.shape[0]
    _, nx, ny, nz = grid.shape

    n_al = -(-n // 8) * 8
    if n_al != n:
        pos = jnp.concatenate([pos, pos[: n_al - n]])

    posx = pos[:, 0]
    posy = pos[:, 1]
    posz = pos[:, 2]

    grid_range = jnp.clip(max_bound - min_bound, 1e-6, None)
    dims = jnp.array([nx - 1, ny - 1, nz - 1], dtype=jnp.float32)
    scales = dims / grid_range
    params = jnp.concatenate(
        [
            jnp.repeat(min_bound.astype(jnp.float32), LANES),
            jnp.repeat(scales.astype(jnp.float32), LANES),
        ]
    )

    table = grid.reshape(-1)

    sc_call = _make_sc_call(n_al, nx, ny, nz)
    outm, outs = sc_call(posx, posy, posz, params, table)
    if n_al != n:
        return (outm[:n], outs[:n])
    return (outm, outs)
